# baseline (speedup 1.0000x reference)
import functools

import jax
import jax.numpy as jnp
from jax import lax
from jax.experimental import pallas as pl
from jax.experimental.pallas import tpu as pltpu
from jax.experimental.pallas import tpu_sc as plsc

_B, _T, _H, _F, _NB, _MAXLEN = 16, 512, 256, 256, 256, 2048
_NZ = 16
_TAUG = _T + _NZ


def _dot(a, b):
    return lax.dot_general(a, b, (((1,), (0,)), ((), ())),
                           preferred_element_type=jnp.float32)


def _embed_body(xref, ptref, etref, durref, eref, binsref, wdep, vdep,
                x2ref, x3ref, idxref, melref):
    del wdep, vdep
    b = pl.program_id(0)
    x = xref[0]
    pt = ptref[0]
    et = etref[0]
    dur_l = durref[0]

    def bucket_embed(v_s, brow, table):
        idx = jnp.sum((brow < v_s).astype(jnp.int32), axis=1, keepdims=True)
        lanes = lax.broadcasted_iota(jnp.int32, (_T, _NB), 1)
        onehot = (lanes == idx).astype(jnp.float32)
        return _dot(onehot, table)

    x2 = x + bucket_embed(pt, binsref[0:1, :], eref[0:_NB, :])
    x3 = x2 + bucket_embed(et, binsref[1:2, :], eref[_NB:2 * _NB, :])
    x2ref[0] = x2.astype(jnp.bfloat16)
    x3ref[...] = jnp.concatenate([x3, jnp.zeros((_NZ, _H), jnp.float32)],
                                 axis=0)

    jl = lax.broadcasted_iota(jnp.int32, (_T, _T), 1)
    ts = lax.broadcasted_iota(jnp.int32, (_T, _T), 0)
    cum_s = jnp.sum(jnp.where(jl <= ts, dur_l, 0), axis=1, keepdims=True)

    t_out = lax.broadcasted_iota(jnp.int32, (1, _MAXLEN), 1)
    idxo = jnp.sum((cum_s <= t_out).astype(jnp.int32), axis=0, keepdims=True)
    cumlast = cum_s[_T - 1:_T, :]
    pad_row = _T + (t_out & (_NZ - 1))
    idx_row = b * _TAUG + jnp.where(t_out < cumlast, idxo, pad_row)
    idxref[...] = idx_row.reshape(_MAXLEN // 128, 128)
    melref[0] = jnp.broadcast_to(cumlast, (1, 128))


def _embed_call(x, pt3, et3, dur3, eflat, bins, wflat, vflat):
    return pl.pallas_call(
        _embed_body,
        grid=(_B,),
        in_specs=[
            pl.BlockSpec((1, _T, _H), lambda b: (b, 0, 0)),
            pl.BlockSpec((1, _T, 1), lambda b: (b, 0, 0)),
            pl.BlockSpec((1, _T, 1), lambda b: (b, 0, 0)),
            pl.BlockSpec((1, 1, _T), lambda b: (b, 0, 0)),
            pl.BlockSpec((2 * _NB, _H), lambda b: (0, 0)),
            pl.BlockSpec((2, _NB), lambda b: (0, 0)),
            pl.BlockSpec((8, 128), lambda b: (0, 0)),
            pl.BlockSpec((8, 128), lambda b: (0, 0)),
        ],
        out_specs=[
            pl.BlockSpec((1, _T, _H), lambda b: (b, 0, 0)),
            pl.BlockSpec((_TAUG, _H), lambda b: (b, 0)),
            pl.BlockSpec((_MAXLEN // 128, 128), lambda b: (b, 0)),
            pl.BlockSpec((1, 1, 128), lambda b: (b, 0, 0)),
        ],
        out_shape=[
            jax.ShapeDtypeStruct((_B, _T, _H), jnp.bfloat16),
            jax.ShapeDtypeStruct((_B * _TAUG, _H), jnp.float32),
            jax.ShapeDtypeStruct((_B * _MAXLEN // 128, 128), jnp.int32),
            jax.ShapeDtypeStruct((_B, 1, 128), jnp.int32),
        ],
        cost_estimate=pl.CostEstimate(
            flops=2 * _B * (2 * _T * _NB * _H + _T * _T + _T * _MAXLEN),
            transcendentals=0, bytes_accessed=40 * 1024 * 1024),
    )(x, pt3, et3, dur3, eflat, bins, wflat, vflat)


_PB = 1
_TP = _PB * _T


def _pred_body(xref, x2ref, keepref, wref, vref, sref, predref):
    x = xref[0]
    x2b = x2ref[0]
    keep = keepref[0]

    def vrow(r):
        return vref[r:r + 1, :]

    def shifts(h):
        z = jnp.zeros((1, _H), h.dtype)
        return (jnp.concatenate([z, h[:-1]], 0), h,
                jnp.concatenate([h[1:], z], 0))

    def conv(h_bf, wbase):
        hm, h0, hp = shifts(h_bf)
        return (_dot(hm, wref[wbase:wbase + _H, :])
                + _dot(h0, wref[wbase + _H:wbase + 2 * _H, :])
                + _dot(hp, wref[wbase + 2 * _H:wbase + 3 * _H, :]))

    def predictor(p, xin_bf):
        ones_col = sref[:, 2 * p + 1:2 * p + 2]
        su_cols = sref[:, 2 * p:2 * p + 2]
        h = jnp.maximum(conv(xin_bf, p * 6 * _H) + vrow(p * 8 + 0), 0.0)
        m = _dot(h, ones_col) * (1.0 / _H)
        v = _dot(h * h, ones_col) * (1.0 / _H) - m * m
        rs = 1.0 / jnp.sqrt(v + 1e-5)
        hn = (((h - m) * rs) * vrow(p * 8 + 1)
              + vrow(p * 8 + 2)).astype(jnp.bfloat16)
        h2 = jnp.maximum(conv(hn, (p * 6 + 3) * _H) + vrow(p * 8 + 3), 0.0)
        both = _dot(h2, su_cols)
        ss2 = _dot(h2 * h2, ones_col)
        su = both[:, 0:1]
        m2 = both[:, 1:2] * (1.0 / _H)
        v2 = ss2 * (1.0 / _H) - m2 * m2
        rs2 = 1.0 / jnp.sqrt(v2 + 1e-5)
        usum = jnp.sum(vrow(p * 8 + 4), axis=1, keepdims=True)[0:1, 0:1]
        c = vrow(p * 8 + 5)[0:1, 0:1]
        out = rs2 * (su - m2 * usum) + c
        return out * keep

    xb = x.astype(jnp.bfloat16)
    cols = [predictor(0, xb), predictor(1, xb), predictor(2, x2b)]
    predref[0] = jnp.concatenate(
        [cols[1], cols[2], cols[0], jnp.zeros((_T, 5), jnp.float32)],
        axis=1)


def _pred_call(x, x2b, keep3, wflat, vflat, scols):
    return pl.pallas_call(
        _pred_body,
        grid=(_B,),
        in_specs=[
            pl.BlockSpec((1, _T, _H), lambda b: (b, 0, 0)),
            pl.BlockSpec((1, _T, _H), lambda b: (b, 0, 0)),
            pl.BlockSpec((1, _T, 1), lambda b: (b, 0, 0)),
            pl.BlockSpec((18 * _H, _F), lambda b: (0, 0)),
            pl.BlockSpec((24, _F), lambda b: (0, 0)),
            pl.BlockSpec((_H, 8), lambda b: (0, 0)),
        ],
        out_specs=[pl.BlockSpec((1, _T, 8), lambda b: (b, 0, 0))],
        out_shape=[jax.ShapeDtypeStruct((_B, _T, 8), jnp.float32)],
        cost_estimate=pl.CostEstimate(
            flops=2 * _B * 6 * _T * 3 * _H * _F,
            transcendentals=0, bytes_accessed=16 * 1024 * 1024),
    )(x, x2b, keep3, wflat, vflat, scols)[0]


_NC, _NS = 2, 16
_NW = _NC * _NS
_ROWS = _B * _MAXLEN
_RPW = _ROWS // _NW
_CH = 128
_NCHUNK = _RPW // _CH
_NBUF = 3


@functools.cache
def _make_sc_gather():
    mesh = plsc.VectorSubcoreMesh(core_axis_name="c", subcore_axis_name="s",
                                  num_cores=_NC, num_subcores=_NS)

    @functools.partial(
        pl.kernel,
        mesh=mesh,
        out_type=jax.ShapeDtypeStruct((_ROWS, _H), jnp.float32),
        cost_estimate=pl.CostEstimate(
            flops=0, transcendentals=0,
            bytes_accessed=2 * _ROWS * _H * 4),
        scratch_types=[
            pltpu.VMEM((_NCHUNK, _CH), jnp.int32),
            [pltpu.VMEM((_CH, _H), jnp.float32) for _ in range(_NBUF)],
            [pltpu.SemaphoreType.DMA for _ in range(_NBUF)],
            [pltpu.SemaphoreType.DMA for _ in range(_NBUF)],
        ],
    )
    def sc_gather(xaug, idx, out, idx_all, rows_v, gsem, ssem):
        wid = lax.axis_index("s") * _NC + lax.axis_index("c")
        gh, sh = {}, {}
        pltpu.sync_copy(idx.at[pl.ds(wid * _NCHUNK, _NCHUNK)], idx_all)

        def off(k):
            return (wid * _NCHUNK + k) * _CH

        def start_gather(k, s):
            gh[k] = pltpu.async_copy(xaug.at[idx_all.at[k]], rows_v[s],
                                     gsem[s])

        for k in range(min(_NBUF, _NCHUNK)):
            start_gather(k, k % _NBUF)
        for k in range(_NCHUNK):
            s = k % _NBUF
            gh[k].wait()
            sh[k] = pltpu.async_copy(
                rows_v[s], out.at[pl.ds(off(k), _CH)], ssem[s])
            if k + _NBUF < _NCHUNK:
                sh[k].wait()
                start_gather(k + _NBUF, s)
        for k in range(max(_NCHUNK - _NBUF, 0), _NCHUNK):
            sh[k].wait()

    return sc_gather


def _sc_gather(xaug, idx):
    return _make_sc_gather()(xaug, idx)


def kernel(x, src_mask, max_len, pitch_target, energy_target, duration_target,
           params, pitch_bins, energy_bins):
    preds = (params['dur'], params['pitch'], params['energy'])
    wflat = jnp.concatenate(
        [p[wn][:, :, k].T for p in preds for wn in ('W1', 'W2')
         for k in range(3)], axis=0).astype(jnp.bfloat16)
    vflat = jnp.stack(
        [r for p in preds
         for r in (p['b1'], p['g1'], p['be1'], p['b2'],
                   p['g2'] * p['Wl'][0],
                   jnp.broadcast_to(jnp.sum(p['be2'] * p['Wl'][0]) + p['bl'][0],
                                    (_F,)),
                   jnp.zeros((_F,), jnp.float32),
                   jnp.zeros((_F,), jnp.float32))], axis=0)
    eflat = jnp.concatenate([params['pitch_emb'], params['energy_emb']], 0)
    inf = jnp.full((1,), jnp.inf, jnp.float32)
    bins = jnp.stack([jnp.concatenate([pitch_bins.astype(jnp.float32), inf]),
                      jnp.concatenate([energy_bins.astype(jnp.float32), inf])])

    keep3 = (1.0 - src_mask.astype(jnp.float32)).reshape(_B, _T, 1)
    pt3 = pitch_target.reshape(_B, _T, 1)
    et3 = energy_target.reshape(_B, _T, 1)
    dur3 = duration_target.astype(jnp.int32).reshape(_B, 1, _T)

    ones = jnp.ones((_F,), jnp.float32)
    scols = jnp.stack([preds[0]['g2'] * preds[0]['Wl'][0], ones,
                       preds[1]['g2'] * preds[1]['Wl'][0], ones,
                       preds[2]['g2'] * preds[2]['Wl'][0], ones,
                       jnp.zeros((_F,), jnp.float32),
                       jnp.zeros((_F,), jnp.float32)], axis=1)

    x2b, x3a, idxg, melb = _embed_call(x, pt3, et3, dur3, eflat, bins,
                                       wflat, vflat)
    out_rows = _sc_gather(x3a, idxg)
    pcols = _pred_call(x, x2b, keep3, wflat, vflat, scols)
    out = out_rows.reshape(_B, _MAXLEN, _H)

    pitch_prediction = pcols[:, :, 0]
    energy_prediction = pcols[:, :, 1]
    log_duration_prediction = pcols[:, :, 2]
    mel_len = jnp.minimum(melb[:, 0, 0], max_len)
    return (out, pitch_prediction, energy_prediction, log_duration_prediction,
            duration_target, mel_len)


# speedup vs baseline: 1.1616x; 1.1616x over previous
import functools

import jax
import jax.numpy as jnp
from jax import lax
from jax.experimental import pallas as pl
from jax.experimental.pallas import tpu as pltpu
from jax.experimental.pallas import tpu_sc as plsc

_B, _T, _H, _F, _NB, _MAXLEN = 16, 512, 256, 256, 256, 2048
_NZ = 16
_TAUG = _T + _NZ


def _dot(a, b):
    return lax.dot_general(a, b, (((1,), (0,)), ((), ())),
                           preferred_element_type=jnp.float32)


def _embed_body(xref, ptref, etref, durref, eref, binsref, wdep, vdep,
                xbref, x2ref, x3ref, idxref, melref):
    del wdep, vdep
    b = pl.program_id(0)
    x = xref[0]
    pt = ptref[0]
    et = etref[0]
    dur_l = durref[0]

    def bucket_embed(v_s, brow, table):
        idx = jnp.sum((brow < v_s).astype(jnp.int32), axis=1, keepdims=True)
        lanes = lax.broadcasted_iota(jnp.int32, (_T, _NB), 1)
        onehot = (lanes == idx).astype(jnp.float32)
        return _dot(onehot, table)

    x2 = x + bucket_embed(pt, binsref[0:1, :], eref[0:_NB, :])
    x3 = x2 + bucket_embed(et, binsref[1:2, :], eref[_NB:2 * _NB, :])
    xbref[0] = x.astype(jnp.bfloat16)
    x2ref[0] = x2.astype(jnp.bfloat16)
    x3ref[...] = jnp.concatenate([x3, jnp.zeros((_NZ, _H), jnp.float32)],
                                 axis=0)

    jl = lax.broadcasted_iota(jnp.int32, (_T, _T), 1)
    ts = lax.broadcasted_iota(jnp.int32, (_T, _T), 0)
    cum_s = jnp.sum(jnp.where(jl <= ts, dur_l, 0), axis=1, keepdims=True)

    t_out = lax.broadcasted_iota(jnp.int32, (1, _MAXLEN), 1)
    idxo = jnp.sum((cum_s <= t_out).astype(jnp.int32), axis=0, keepdims=True)
    cumlast = cum_s[_T - 1:_T, :]
    pad_row = _T + (t_out & (_NZ - 1))
    idx_row = b * _TAUG + jnp.where(t_out < cumlast, idxo, pad_row)
    idxref[...] = idx_row.reshape(_MAXLEN // 128, 128)
    melref[0] = jnp.broadcast_to(cumlast, (1, 128))


def _embed_call(x, pt3, et3, dur3, eflat, bins, wflat, vflat):
    return pl.pallas_call(
        _embed_body,
        grid=(_B,),
        in_specs=[
            pl.BlockSpec((1, _T, _H), lambda b: (b, 0, 0)),
            pl.BlockSpec((1, _T, 1), lambda b: (b, 0, 0)),
            pl.BlockSpec((1, _T, 1), lambda b: (b, 0, 0)),
            pl.BlockSpec((1, 1, _T), lambda b: (b, 0, 0)),
            pl.BlockSpec((2 * _NB, _H), lambda b: (0, 0)),
            pl.BlockSpec((2, _NB), lambda b: (0, 0)),
            pl.BlockSpec((8, 128), lambda b: (0, 0)),
            pl.BlockSpec((8, 128), lambda b: (0, 0)),
        ],
        out_specs=[
            pl.BlockSpec((1, _T, _H), lambda b: (b, 0, 0)),
            pl.BlockSpec((1, _T, _H), lambda b: (b, 0, 0)),
            pl.BlockSpec((_TAUG, _H), lambda b: (b, 0)),
            pl.BlockSpec((_MAXLEN // 128, 128), lambda b: (b, 0)),
            pl.BlockSpec((1, 1, 128), lambda b: (b, 0, 0)),
        ],
        out_shape=[
            jax.ShapeDtypeStruct((_B, _T, _H), jnp.bfloat16),
            jax.ShapeDtypeStruct((_B, _T, _H), jnp.bfloat16),
            jax.ShapeDtypeStruct((_B * _TAUG, _H), jnp.float32),
            jax.ShapeDtypeStruct((_B * _MAXLEN // 128, 128), jnp.int32),
            jax.ShapeDtypeStruct((_B, 1, 128), jnp.int32),
        ],
        cost_estimate=pl.CostEstimate(
            flops=2 * _B * (2 * _T * _NB * _H + _T * _T + _T * _MAXLEN),
            transcendentals=0, bytes_accessed=40 * 1024 * 1024),
    )(x, pt3, et3, dur3, eflat, bins, wflat, vflat)


_PB = 1
_TP = _PB * _T


def _pred_body(xbref, x2ref, keepref, wref, vref, predref):
    xb = xbref[0]
    x2b = x2ref[0]
    keep = keepref[0]

    def vrow(r):
        return vref[r:r + 1, :]

    def shifts(h):
        z = jnp.zeros((1, _H), h.dtype)
        return (jnp.concatenate([z, h[:-1]], 0), h,
                jnp.concatenate([h[1:], z], 0))

    def conv(h_bf, wbase):
        hm, h0, hp = shifts(h_bf)
        return (_dot(hm, wref[wbase:wbase + _H, :])
                + _dot(h0, wref[wbase + _H:wbase + 2 * _H, :])
                + _dot(hp, wref[wbase + 2 * _H:wbase + 3 * _H, :]))

    def predictor(p, xin_bf):
        h = jnp.maximum(conv(xin_bf, p * 6 * _H) + vrow(p * 8 + 0), 0.0)
        m = jnp.mean(h, axis=1, keepdims=True)
        v = jnp.mean(h * h, axis=1, keepdims=True) - m * m
        rs = 1.0 / jnp.sqrt(v + 1e-5)
        hn = (((h - m) * rs) * vrow(p * 8 + 1)
              + vrow(p * 8 + 2)).astype(jnp.bfloat16)
        h2 = jnp.maximum(conv(hn, (p * 6 + 3) * _H) + vrow(p * 8 + 3), 0.0)
        u = vrow(p * 8 + 4)
        m2 = jnp.mean(h2, axis=1, keepdims=True)
        v2 = jnp.mean(h2 * h2, axis=1, keepdims=True) - m2 * m2
        rs2 = 1.0 / jnp.sqrt(v2 + 1e-5)
        su = jnp.sum(h2 * u, axis=1, keepdims=True)
        usum = jnp.sum(u, axis=1, keepdims=True)[0:1, 0:1]
        c = vrow(p * 8 + 5)[0:1, 0:1]
        out = rs2 * (su - m2 * usum) + c
        return out * keep

    cols = [predictor(0, xb), predictor(1, xb), predictor(2, x2b)]
    predref[0] = jnp.concatenate(
        [cols[1], cols[2], cols[0], jnp.zeros((_T, 5), jnp.float32)],
        axis=1)


def _pred_call(xb, x2b, keep3, wflat, vflat):
    return pl.pallas_call(
        _pred_body,
        grid=(_B,),
        in_specs=[
            pl.BlockSpec((1, _T, _H), lambda b: (b, 0, 0)),
            pl.BlockSpec((1, _T, _H), lambda b: (b, 0, 0)),
            pl.BlockSpec((1, _T, 1), lambda b: (b, 0, 0)),
            pl.BlockSpec((18 * _H, _F), lambda b: (0, 0)),
            pl.BlockSpec((24, _F), lambda b: (0, 0)),
        ],
        out_specs=[pl.BlockSpec((1, _T, 8), lambda b: (b, 0, 0))],
        out_shape=[jax.ShapeDtypeStruct((_B, _T, 8), jnp.float32)],
        cost_estimate=pl.CostEstimate(
            flops=2 * _B * 6 * _T * 3 * _H * _F,
            transcendentals=0, bytes_accessed=16 * 1024 * 1024),
    )(xb, x2b, keep3, wflat, vflat)[0]


_NC, _NS = 2, 16
_NW = _NC * _NS
_ROWS = _B * _MAXLEN
_RPW = _ROWS // _NW
_CH = 128
_NCHUNK = _RPW // _CH
_NBUF = 3


@functools.cache
def _make_sc_gather():
    mesh = plsc.VectorSubcoreMesh(core_axis_name="c", subcore_axis_name="s",
                                  num_cores=_NC, num_subcores=_NS)

    @functools.partial(
        pl.kernel,
        mesh=mesh,
        out_type=jax.ShapeDtypeStruct((_ROWS, _H), jnp.float32),
        cost_estimate=pl.CostEstimate(
            flops=0, transcendentals=0,
            bytes_accessed=2 * _ROWS * _H * 4),
        scratch_types=[
            pltpu.VMEM((_NCHUNK, _CH), jnp.int32),
            [pltpu.VMEM((_CH, _H), jnp.float32) for _ in range(_NBUF)],
            [pltpu.SemaphoreType.DMA for _ in range(_NBUF)],
            [pltpu.SemaphoreType.DMA for _ in range(_NBUF)],
        ],
    )
    def sc_gather(xaug, idx, out, idx_all, rows_v, gsem, ssem):
        wid = lax.axis_index("s") * _NC + lax.axis_index("c")
        gh, sh = {}, {}
        pltpu.sync_copy(idx.at[pl.ds(wid * _NCHUNK, _NCHUNK)], idx_all)

        def off(k):
            return (wid * _NCHUNK + k) * _CH

        def start_gather(k, s):
            gh[k] = pltpu.async_copy(xaug.at[idx_all.at[k]], rows_v[s],
                                     gsem[s])

        for k in range(min(_NBUF, _NCHUNK)):
            start_gather(k, k % _NBUF)
        for k in range(_NCHUNK):
            s = k % _NBUF
            gh[k].wait()
            sh[k] = pltpu.async_copy(
                rows_v[s], out.at[pl.ds(off(k), _CH)], ssem[s])
            if k + _NBUF < _NCHUNK:
                sh[k].wait()
                start_gather(k + _NBUF, s)
        for k in range(max(_NCHUNK - _NBUF, 0), _NCHUNK):
            sh[k].wait()

    return sc_gather


def _sc_gather(xaug, idx):
    return _make_sc_gather()(xaug, idx)


def kernel(x, src_mask, max_len, pitch_target, energy_target, duration_target,
           params, pitch_bins, energy_bins):
    preds = (params['dur'], params['pitch'], params['energy'])
    wflat = jnp.concatenate(
        [p[wn][:, :, k].T for p in preds for wn in ('W1', 'W2')
         for k in range(3)], axis=0).astype(jnp.bfloat16)
    vflat = jnp.stack(
        [r for p in preds
         for r in (p['b1'], p['g1'], p['be1'], p['b2'],
                   p['g2'] * p['Wl'][0],
                   jnp.broadcast_to(jnp.sum(p['be2'] * p['Wl'][0]) + p['bl'][0],
                                    (_F,)),
                   jnp.zeros((_F,), jnp.float32),
                   jnp.zeros((_F,), jnp.float32))], axis=0)
    eflat = jnp.concatenate([params['pitch_emb'], params['energy_emb']], 0)
    inf = jnp.full((1,), jnp.inf, jnp.float32)
    bins = jnp.stack([jnp.concatenate([pitch_bins.astype(jnp.float32), inf]),
                      jnp.concatenate([energy_bins.astype(jnp.float32), inf])])

    keep3 = (1.0 - src_mask.astype(jnp.float32)).reshape(_B, _T, 1)
    pt3 = pitch_target.reshape(_B, _T, 1)
    et3 = energy_target.reshape(_B, _T, 1)
    dur3 = duration_target.astype(jnp.int32).reshape(_B, 1, _T)

    xb, x2b, x3a, idxg, melb = _embed_call(x, pt3, et3, dur3, eflat, bins,
                                           wflat, vflat)
    out_rows = _sc_gather(x3a, idxg)
    pcols = _pred_call(xb, x2b, keep3, wflat, vflat)
    out = out_rows.reshape(_B, _MAXLEN, _H)

    pitch_prediction = pcols[:, :, 0]
    energy_prediction = pcols[:, :, 1]
    log_duration_prediction = pcols[:, :, 2]
    mel_len = jnp.minimum(melb[:, 0, 0], max_len)
    return (out, pitch_prediction, energy_prediction, log_duration_prediction,
            duration_target, mel_len)
